# Initial kernel scaffold; baseline (speedup 1.0000x reference)
import jax
import jax.numpy as jnp
from jax.experimental import pallas as pl


def kernel(x, edge_index, edge_attr, graph_ids, atom_emb, vn_emb, bond_emb, eps, gin_W1, gin_b1, gin_g1, gin_be1, gin_W2, gin_b2, bn_g, bn_be, vn_W1, vn_b1, vn_g1, vn_be1, vn_W2, vn_b2, vn_g2, vn_be2):
    raise NotImplementedError("write your pallas kernel here")


# speedup vs baseline: 2.5040x; 2.5040x over previous
import dataclasses
import functools

import jax
import jax.numpy as jnp
from jax import lax
from jax.experimental import pallas as pl
from jax.experimental.pallas import tpu as pltpu
from jax.experimental.pallas import tpu_sc as plsc

N, E, D, L, B = 10000, 320000, 128, 5, 64
ATOM_V, BOND_V, AF, BF = 100, 8, 9, 3

NC, NS = 2, 16
NW = NC * NS
NP = 10240
BLK = 400
NBLK = N // BLK

BINS = 320
CAP = 16384
SCCH = 512
C = 128


def _mesh():
    return plsc.VectorSubcoreMesh(core_axis_name="c", subcore_axis_name="s",
                                  num_cores=NC, num_subcores=NS)


def _sc_params():
    cp = pltpu.CompilerParams()
    if "needs_layout_passes" in pltpu.CompilerParams.__dataclass_fields__:
        cp = dataclasses.replace(cp, needs_layout_passes=False)
    return cp


def _edge_partition(src, dst, ecode):
    @functools.partial(
        pl.kernel,
        out_type=[
            jax.ShapeDtypeStruct((NW, CAP), jnp.int32),
            jax.ShapeDtypeStruct((NW, CAP), jnp.int32),
            jax.ShapeDtypeStruct((NW, 16), jnp.int32),
        ],
        mesh=_mesh(),
        compiler_params=_sc_params(),
        scratch_types=[
            pltpu.VMEM((1, SCCH), jnp.int32),
            pltpu.VMEM((1, SCCH), jnp.int32),
            pltpu.VMEM((1, SCCH), jnp.int32),
            pltpu.VMEM((CAP,), jnp.int32),
            pltpu.VMEM((CAP,), jnp.int32),
            pltpu.VMEM((1, 16), jnp.int32),
        ],
    )
    def k(src_hbm, dst_hbm, code_hbm, sl_hbm, pk_hbm, cnt_hbm,
          sv, dv, cv, slb, pkb, cntv):
        cid = lax.axis_index("c")
        sid = lax.axis_index("s")
        wid = cid * NS + sid
        lo = wid * BINS
        hi = lo + BINS

        @pl.loop(0, CAP // 16)
        def _(j):
            s_ = pl.ds(j * 16, 16)
            slb[s_] = jnp.zeros((16,), jnp.int32)
            pkb[s_] = jnp.full((16,), BINS * 512, jnp.int32)

        def chunk(i, pos):
            eb = i * SCCH
            pltpu.sync_copy(src_hbm.at[pl.ds(eb, SCCH)], sv.at[0])
            pltpu.sync_copy(dst_hbm.at[pl.ds(eb, SCCH)], dv.at[0])
            pltpu.sync_copy(code_hbm.at[pl.ds(eb, SCCH)], cv.at[0])

            def vec(j, pos):
                s_ = pl.ds(j * 16, 16)
                dvec = dv[0, s_]
                msk = (dvec >= lo) & (dvec < hi)
                packed = (dvec - lo) * 512 + cv[0, s_]
                plsc.store_compressed(slb.at[pl.ds(pos, 16)], sv[0, s_], mask=msk)
                plsc.store_compressed(pkb.at[pl.ds(pos, 16)], packed, mask=msk)
                return pos + jnp.sum(msk.astype(jnp.int32))

            return lax.fori_loop(0, SCCH // 16, vec, pos)

        pos = lax.fori_loop(0, E // SCCH, chunk, jnp.int32(0))
        cntv[0, :] = jnp.zeros((16,), jnp.int32) + pos
        pltpu.sync_copy(cntv.at[0], cnt_hbm.at[wid])

        @pl.loop(0, CAP // SCCH)
        def _(j):
            s_ = pl.ds(j * SCCH, SCCH)
            pltpu.sync_copy(slb.at[s_], sl_hbm.at[wid, s_])
            pltpu.sync_copy(pkb.at[s_], pk_hbm.at[wid, s_])

    return k(src, dst, ecode)


def _edge_pass_sc(h_in, sl, pk, cnt, comb):
    @functools.partial(
        pl.kernel,
        out_type=jax.ShapeDtypeStruct((NP, D), jnp.float32),
        mesh=_mesh(),
        scratch_types=[
            pltpu.VMEM((1, C), jnp.int32),
            pltpu.VMEM((1, C), jnp.int32),
            pltpu.VMEM((C, D), jnp.float32),
            pltpu.VMEM((C, D), jnp.float32),
            pltpu.VMEM((BINS + 8, D), jnp.float32),
            pltpu.VMEM((1, C), jnp.int32),
            pltpu.VMEM((1, 16), jnp.int32),
            pltpu.SemaphoreType.DMA,
            pltpu.SemaphoreType.DMA,
        ],
    )
    def k(h_hbm, sl_hbm, pk_hbm, cnt_hbm, comb_hbm, out_hbm,
          srcv, codev, hrows, erows, agg, dsts, cnts, sem1, sem2):
        cid = lax.axis_index("c")
        sid = lax.axis_index("s")
        wid = cid * NS + sid
        lo = wid * BINS

        @pl.loop(0, BINS + 8)
        def _(r):
            for c8 in range(D // 16):
                agg[r, pl.ds(c8 * 16, 16)] = jnp.zeros((16,), jnp.float32)

        pltpu.sync_copy(cnt_hbm.at[wid], cnts.at[0])
        n = cnts[0, pl.ds(0, 16)][0]
        nch = (n + C - 1) // C

        def chunk(i, _):
            s_ = pl.ds(i * C, C)
            pltpu.sync_copy(sl_hbm.at[wid, s_], srcv.at[0])
            pltpu.sync_copy(pk_hbm.at[wid, s_], dsts.at[0])
            for j in range(C // 16):
                cs = pl.ds(j * 16, 16)
                codev[0, cs] = jnp.bitwise_and(dsts[0, cs], 511)
            cp1 = pltpu.async_copy(h_hbm.at[srcv.at[0]], hrows, sem1)
            cp2 = pltpu.async_copy(comb_hbm.at[codev.at[0]], erows, sem2)
            cp1.wait()
            cp2.wait()

            @pl.loop(0, C)
            def _(r):
                for c8 in range(D // 16):
                    cs = pl.ds(c8 * 16, 16)
                    hrows[r, cs] = jnp.maximum(hrows[r, cs] + erows[r, cs],
                                               0.0)

            @pl.loop(0, C // 16)
            def _(g):
                dvec = lax.shift_right_logical(dsts[0, pl.ds(g * 16, 16)], 9)
                for j in range(16):
                    d = dvec[j]
                    for c8 in range(D // 16):
                        cs = pl.ds(c8 * 16, 16)
                        agg[d, cs] += hrows[g * 16 + j, cs]

            return 0

        lax.fori_loop(0, nch, chunk, 0)

        @pl.loop(0, (BINS + C - 1) // C)
        def _(j):
            r0 = j * C
            pltpu.sync_copy(agg.at[pl.ds(r0, C)],
                            out_hbm.at[pl.ds(lo + r0, C)])

    return k(h_in, sl, pk, cnt, comb)


def _pool_bounds(gids):
    @functools.partial(
        pl.kernel,
        out_type=jax.ShapeDtypeStruct((NW, 16), jnp.int32),
        mesh=_mesh(),
        compiler_params=_sc_params(),
        scratch_types=[
            pltpu.VMEM((1, 400), jnp.int32),
            pltpu.VMEM((1, 16), jnp.int32),
        ],
    )
    def k(g_hbm, out_hbm, gv, bv):
        cid = lax.axis_index("c")
        sid = lax.axis_index("s")
        wid = cid * NS + sid
        g0 = wid * 2

        def count_below(thresh):
            def scan(i, cnt):
                pltpu.sync_copy(g_hbm.at[pl.ds(i * 400, 400)], gv.at[0])

                def vec(j, cnt):
                    v = gv[0, pl.ds(j * 16, 16)]
                    return cnt + jnp.sum((v < thresh).astype(jnp.int32))

                return lax.fori_loop(0, 400 // 16, vec, cnt)

            return lax.fori_loop(0, N // 400, scan, jnp.int32(0))

        clo = count_below(g0)
        cmid = count_below(g0 + 1)
        chi = count_below(g0 + 2)
        z = lax.iota(jnp.int32, 16)
        bv[0, :] = (jnp.where(z == 0, clo, 0) + jnp.where(z == 1, cmid, 0)
                    + jnp.where(z == 2, chi, 0))
        pltpu.sync_copy(bv.at[0], out_hbm.at[wid])

    return k(gids)


def _pool_fold(h_in, bounds):
    @functools.partial(
        pl.kernel,
        out_type=jax.ShapeDtypeStruct((B * D,), jnp.float32),
        mesh=_mesh(),
        scratch_types=[
            pltpu.VMEM((1, 16), jnp.int32),
            pltpu.VMEM((1, C), jnp.int32),
            pltpu.VMEM((C, D), jnp.float32),
            pltpu.VMEM((2, D), jnp.float32),
            pltpu.SemaphoreType.DMA,
        ],
    )
    def k(h_hbm, b_hbm, out_hbm, bv, idxv, rows, acc, sem):
        cid = lax.axis_index("c")
        sid = lax.axis_index("s")
        wid = cid * NS + sid
        g0 = wid * 2

        pltpu.sync_copy(b_hbm.at[wid], bv.at[0])
        bvec = bv[0, pl.ds(0, 16)]
        clo = bvec[0]
        cmid = bvec[1]
        chi = bvec[2]

        for c8 in range(D // 16):
            cs = pl.ds(c8 * 16, 16)
            acc[0, cs] = jnp.zeros((16,), jnp.float32)
            acc[1, cs] = jnp.zeros((16,), jnp.float32)

        def fold(gslot, rlo, rhi):
            nch = (rhi - rlo + C - 1) // C

            def chunk(kk, _):
                base = rlo + kk * C
                for j in range(C // 16):
                    idxv[0, pl.ds(j * 16, 16)] = jnp.minimum(
                        lax.iota(jnp.int32, 16) + (base + j * 16), N - 1)
                pltpu.async_copy(h_hbm.at[idxv.at[0]], rows, sem).wait()
                rem = jnp.minimum(rhi - base, C)

                @pl.loop(0, C)
                def _(r):
                    @pl.when(r < rem)
                    def _():
                        for c8 in range(D // 16):
                            cs = pl.ds(c8 * 16, 16)
                            acc[gslot, cs] += rows[r, cs]

                return 0

            lax.fori_loop(0, nch, chunk, 0)

        fold(0, clo, cmid)
        fold(1, cmid, chi)
        pltpu.sync_copy(acc.at[0], out_hbm.at[pl.ds(g0 * D, D)])
        pltpu.sync_copy(acc.at[1], out_hbm.at[pl.ds((g0 + 1) * D, D)])

    return k(h_in, bounds).reshape(B, D)


def _atom_sc(xc, atom_flat):
    RT = 320
    RC = 64

    @functools.partial(
        pl.kernel,
        out_type=jax.ShapeDtypeStruct((N, D), jnp.float32),
        mesh=_mesh(),
        scratch_types=[
            pltpu.VMEM((1, RC), jnp.int32),
            pltpu.VMEM((1, RC), jnp.int32),
            pltpu.VMEM((RC, D), jnp.float32),
            pltpu.VMEM((RC, D), jnp.float32),
            pltpu.SemaphoreType.DMA,
        ],
    )
    def k(xc_hbm, emb_hbm, out_hbm, xv, codev, grows, hacc, sem):
        cid = lax.axis_index("c")
        sid = lax.axis_index("s")
        wid = cid * NS + sid

        @pl.loop(0, RT // RC)
        def _(ck):
            row0 = jnp.minimum(wid * RT + ck * RC, N - RC)

            @pl.loop(0, RC)
            def _(r):
                for c8 in range(D // 16):
                    hacc[r, pl.ds(c8 * 16, 16)] = jnp.zeros((16,),
                                                            jnp.float32)

            for i in range(AF):
                pltpu.sync_copy(xc_hbm.at[pl.ds(i * N + row0, RC)], xv.at[0])
                for j in range(RC // 16):
                    cs = pl.ds(j * 16, 16)
                    codev[0, cs] = xv[0, cs] + (100 * i)
                pltpu.async_copy(emb_hbm.at[codev.at[0]], grows, sem).wait()

                @pl.loop(0, RC)
                def _(r):
                    for c8 in range(D // 16):
                        cs = pl.ds(c8 * 16, 16)
                        hacc[r, cs] += grows[r, cs]

            pltpu.sync_copy(hacc, out_hbm.at[pl.ds(row0, RC)])

    return k(xc, atom_flat)


def _comb_tables(bond_emb):
    def body(be_ref, out_ref):
        b = be_ref[0]
        s = (b[0][:, None, None, :] + b[1][None, :, None, :]
             + b[2][None, None, :, :])
        out_ref[0] = s.reshape(BOND_V ** BF, D)

    return pl.pallas_call(
        body,
        grid=(L,),
        in_specs=[pl.BlockSpec((1, BF, BOND_V, D), lambda l: (l, 0, 0, 0))],
        out_specs=pl.BlockSpec((1, BOND_V ** BF, D), lambda l: (l, 0, 0)),
        out_shape=jax.ShapeDtypeStruct((L, BOND_V ** BF, D), jnp.float32),
    )(bond_emb)


def _pre_layer(h, vn, gids3):
    def body(h_ref, g_ref, vn_ref, hin_ref):
        gid = g_ref[0, 0, :]
        colb = lax.broadcasted_iota(jnp.int32, (BLK, B), 1)
        oh = (gid[:, None] == colb).astype(jnp.float32)
        hin_ref[...] = h_ref[...] + jnp.dot(
            oh, vn_ref[...], preferred_element_type=jnp.float32,
            precision=lax.Precision.HIGHEST)

    return pl.pallas_call(
        body,
        grid=(NBLK,),
        in_specs=[
            pl.BlockSpec((BLK, D), lambda i: (i, 0)),
            pl.BlockSpec((1, 1, BLK), lambda i: (i, 0, 0)),
            pl.BlockSpec((B, D), lambda i: (0, 0)),
        ],
        out_specs=pl.BlockSpec((BLK, D), lambda i: (i, 0)),
        out_shape=jax.ShapeDtypeStruct((N, D), jnp.float32),
    )(h, gids3, vn)


def _col_stats(u):
    def body(u_ref, s_ref):
        p = pl.program_id(0)
        i = pl.program_id(1)

        @pl.when((p == 0) & (i == 0))
        def _():
            s_ref[...] = jnp.zeros((8, D), jnp.float32)

        @pl.when(p == 0)
        def _():
            s_ref[0, :] += jnp.sum(u_ref[...], axis=0)

        @pl.when(p == 1)
        def _():
            mu = s_ref[0, :] / N
            du = u_ref[...] - mu[None, :]
            s_ref[1, :] += jnp.sum(du * du, axis=0)

    return pl.pallas_call(
        body,
        grid=(2, NBLK),
        in_specs=[pl.BlockSpec((BLK, D), lambda p, i: (i, 0))],
        out_specs=pl.BlockSpec((8, D), lambda p, i: (0, 0)),
        out_shape=jax.ShapeDtypeStruct((8, D), jnp.float32),
    )(u)


def _mlp_stage1(h_in, agg, scale, W1, b1):
    def body(hin_ref, a_ref, sc_ref, w_ref, b_ref, u_ref):
        t0 = hin_ref[...] * sc_ref[0, 0] + a_ref[...]
        u_ref[...] = jnp.dot(t0, w_ref[...],
                             preferred_element_type=jnp.float32) + b_ref[...]

    return pl.pallas_call(
        body,
        grid=(NBLK,),
        in_specs=[
            pl.BlockSpec((BLK, D), lambda i: (i, 0)),
            pl.BlockSpec((BLK, D), lambda i: (i, 0)),
            pl.BlockSpec((1, 1), lambda i: (0, 0), memory_space=pltpu.SMEM),
            pl.BlockSpec((D, D), lambda i: (0, 0)),
            pl.BlockSpec((1, D), lambda i: (0, 0)),
        ],
        out_specs=pl.BlockSpec((BLK, D), lambda i: (i, 0)),
        out_shape=jax.ShapeDtypeStruct((N, D), jnp.float32),
    )(h_in, agg, scale, W1, b1)


def _mlp_stage2(u1, s1, g1, be1, W2, b2):
    def body(u_ref, s1_ref, g_ref, be_ref, w_ref, b_ref, u2_ref):
        mu = s1_ref[0, :] / N
        var = s1_ref[1, :] / N
        rs = lax.rsqrt(var + 1e-5)
        y = ((u_ref[...] - mu[None, :]) * rs[None, :]) * g_ref[...] \
            + be_ref[...]
        y = jnp.maximum(y, 0.0)
        u2_ref[...] = jnp.dot(y, w_ref[...],
                              preferred_element_type=jnp.float32) + b_ref[...]

    return pl.pallas_call(
        body,
        grid=(NBLK,),
        in_specs=[
            pl.BlockSpec((BLK, D), lambda i: (i, 0)),
            pl.BlockSpec((8, D), lambda i: (0, 0)),
            pl.BlockSpec((1, D), lambda i: (0, 0)),
            pl.BlockSpec((1, D), lambda i: (0, 0)),
            pl.BlockSpec((D, D), lambda i: (0, 0)),
            pl.BlockSpec((1, D), lambda i: (0, 0)),
        ],
        out_specs=pl.BlockSpec((BLK, D), lambda i: (i, 0)),
        out_shape=jax.ShapeDtypeStruct((N, D), jnp.float32),
    )(u1, s1, g1, be1, W2, b2)


def _mlp_stage3(u2, s2, g, be, relu):
    def body(u_ref, s_ref, g_ref, be_ref, out_ref):
        mu = s_ref[0, :] / N
        var = s_ref[1, :] / N
        rs = lax.rsqrt(var + 1e-5)
        t = ((u_ref[...] - mu[None, :]) * rs[None, :]) * g_ref[...] \
            + be_ref[...]
        if relu:
            t = jnp.maximum(t, 0.0)
        out_ref[...] = t

    return pl.pallas_call(
        body,
        grid=(NBLK,),
        in_specs=[
            pl.BlockSpec((BLK, D), lambda i: (i, 0)),
            pl.BlockSpec((8, D), lambda i: (0, 0)),
            pl.BlockSpec((1, D), lambda i: (0, 0)),
            pl.BlockSpec((1, D), lambda i: (0, 0)),
        ],
        out_specs=pl.BlockSpec((BLK, D), lambda i: (i, 0)),
        out_shape=jax.ShapeDtypeStruct((N, D), jnp.float32),
    )(u2, s2, g, be)


def _vn_mlp(pool, vn, W1, b1, g1, be1, W2, b2, g2, be2):
    def body(p_ref, vn_ref, w1_ref, b1_ref, g1_ref, be1_ref,
             w2_ref, b2_ref, g2_ref, be2_ref, out_ref):
        v = p_ref[...] + vn_ref[...]
        a = jnp.dot(v, w1_ref[...],
                    preferred_element_type=jnp.float32) + b1_ref[...]
        mu = jnp.mean(a, axis=0)
        var = jnp.mean((a - mu[None, :]) ** 2, axis=0)
        rs = lax.rsqrt(var + 1e-5)
        a = ((a - mu[None, :]) * rs[None, :]) * g1_ref[...] + be1_ref[...]
        a = jnp.maximum(a, 0.0)
        c = jnp.dot(a, w2_ref[...],
                    preferred_element_type=jnp.float32) + b2_ref[...]
        mu2 = jnp.mean(c, axis=0)
        var2 = jnp.mean((c - mu2[None, :]) ** 2, axis=0)
        rs2 = lax.rsqrt(var2 + 1e-5)
        c = ((c - mu2[None, :]) * rs2[None, :]) * g2_ref[...] + be2_ref[...]
        out_ref[...] = jnp.maximum(c, 0.0)

    return pl.pallas_call(
        body,
        out_shape=jax.ShapeDtypeStruct((B, D), jnp.float32),
    )(pool, vn, W1, b1, g1, be1, W2, b2, g2, be2)


def kernel(x, edge_index, edge_attr, graph_ids, atom_emb, vn_emb, bond_emb,
           eps, gin_W1, gin_b1, gin_g1, gin_be1, gin_W2, gin_b2, bn_g, bn_be,
           vn_W1, vn_b1, vn_g1, vn_be1, vn_W2, vn_b2, vn_g2, vn_be2):
    f32 = jnp.float32
    src = edge_index[0].astype(jnp.int32)
    dst = edge_index[1].astype(jnp.int32)
    ea = edge_attr.astype(jnp.int32)
    ecode = ea[:, 0] * (BOND_V * BOND_V) + ea[:, 1] * BOND_V + ea[:, 2]
    xc = x.astype(jnp.int32).T.reshape(AF * N)
    gids = graph_ids.astype(jnp.int32)
    gids3 = gids.reshape(NBLK, 1, BLK)
    atom_flat = atom_emb.astype(f32).reshape(AF * ATOM_V, D)
    vn0 = jnp.broadcast_to(vn_emb[0].astype(f32), (B, D))
    eps_s = (1.0 + eps.astype(f32)).reshape(L, 1)
    b1r = gin_b1.reshape(L, 1, D)
    b2r = gin_b2.reshape(L, 1, D)

    comb = _comb_tables(bond_emb.astype(f32))
    sl, pk, cnt = _edge_partition(src, dst, ecode)
    gbounds = _pool_bounds(gids)
    h = _atom_sc(xc, atom_flat)
    vn = vn0

    for l in range(L):
        h_in = _pre_layer(h, vn, gids3)
        agg = _edge_pass_sc(h_in, sl, pk, cnt, comb[l])
        if l < L - 1:
            pool = _pool_fold(h_in, gbounds)
        u1 = _mlp_stage1(h_in, agg[:N], eps_s[l].reshape(1, 1),
                         gin_W1[l], b1r[l])
        s1 = _col_stats(u1)
        u2 = _mlp_stage2(u1, s1, gin_g1[l].reshape(1, D),
                         gin_be1[l].reshape(1, D), gin_W2[l], b2r[l])
        s2 = _col_stats(u2)
        h = _mlp_stage3(u2, s2, bn_g[l].reshape(1, D),
                        bn_be[l].reshape(1, D), relu=(l < L - 1))
        if l < L - 1:
            vn = _vn_mlp(pool, vn, vn_W1[l], vn_b1[l].reshape(1, D),
                         vn_g1[l].reshape(1, D), vn_be1[l].reshape(1, D),
                         vn_W2[l], vn_b2[l].reshape(1, D),
                         vn_g2[l].reshape(1, D), vn_be2[l].reshape(1, D))
    return h
